# baseline (speedup 1.0000x reference)
import functools

import jax
import jax.numpy as jnp
import numpy as np
from jax.experimental import pallas as pl
from jax.experimental.pallas import tpu as pltpu

H = 4096
E = 16
TOP_K = 2
B = 4
S = 2048
N = B * S
HH = H // 2
TEMPERATURE = 0.7
CAPACITY_FACTOR = 2.0
EPS = 1e-6
CAPACITY = int(CAPACITY_FACTOR * B * S * TOP_K / E)

TB = 512
NB = 256
NBLK = HH // NB


def _mlp_kernel(x_ref, lnw_ref, lnb_ref, w1_ref, b1_ref, w2_ref, b2_ref,
                out_ref, xn_ref, acc_ref):
    j = pl.program_id(1)

    @pl.when(j == 0)
    def _ln():
        x = x_ref[...]
        mu = jnp.mean(x, axis=1, keepdims=True)
        xc = x - mu
        var = jnp.mean(xc * xc, axis=1, keepdims=True)
        inv = 1.0 / jnp.sqrt(var + 1e-5)
        xn_ref[...] = (xc * inv) * lnw_ref[...] + lnb_ref[...]

    h = jnp.dot(xn_ref[...], w1_ref[...], preferred_element_type=jnp.float32)
    h = h + b1_ref[...]
    h = 0.5 * h * (1.0 + jax.lax.erf(h * (1.0 / np.sqrt(2.0))))
    part = jnp.dot(h, w2_ref[...], preferred_element_type=jnp.float32)

    @pl.when(j == 0)
    def _init():
        acc_ref[...] = part

    @pl.when(j > 0)
    def _acc():
        acc_ref[...] = acc_ref[...] + part

    @pl.when(j == NBLK - 1)
    def _emit():
        ew = (acc_ref[...] + b2_ref[...]) / TEMPERATURE
        out_ref[...] = jnp.clip(ew, -50.0, 50.0)


def _routing_kernel(ew_ref, masks_ref, usage_ref, loss_ref):
    ew = ew_ref[...]
    idx_e = jax.lax.broadcasted_iota(jnp.int32, (N, E), 1)
    m1 = jnp.max(ew, axis=1, keepdims=True)
    i1 = jnp.min(jnp.where(ew == m1, idx_e, E), axis=1, keepdims=True)
    ew_m = jnp.where(idx_e == i1, -1e30, ew)
    m2 = jnp.max(ew_m, axis=1, keepdims=True)
    i2 = jnp.min(jnp.where(ew_m == m2, idx_e, E), axis=1, keepdims=True)
    e2 = jnp.exp(m2 - m1)
    denom = 1.0 + e2
    p1 = 1.0 / denom
    p2 = e2 / denom
    masks = jnp.where(idx_e == i1, p1, 0.0) + jnp.where(idx_e == i2, p2, 0.0)

    col_sums = jnp.sum(masks, axis=0, keepdims=True)
    masks_ref[...] = masks

    @pl.when(jnp.any(col_sums > CAPACITY))
    def _cap():
        vbits = jax.lax.bitcast_convert_type(masks, jnp.int32)
        thr = jnp.zeros((1, E), jnp.int32)
        for b in range(29, -1, -1):
            cand = thr | (1 << b)
            cnt = jnp.sum((vbits >= cand).astype(jnp.int32), axis=0,
                          keepdims=True)
            thr = jnp.where(cnt >= CAPACITY, cand, thr)
        count_gt = jnp.sum((vbits > thr).astype(jnp.int32), axis=0,
                           keepdims=True)
        rem = CAPACITY - count_gt
        eq = vbits == thr
        idx_n = jax.lax.broadcasted_iota(jnp.int32, (N, E), 0)
        jthr = jnp.zeros((1, E), jnp.int32)
        for b in range(12, -1, -1):
            cand = jthr | (1 << b)
            cnt = jnp.sum((eq & (idx_n <= cand)).astype(jnp.int32), axis=0,
                          keepdims=True)
            jthr = jnp.where(cnt <= rem, cand, jthr)
        keep = (vbits > thr) | (eq & (idx_n <= jthr))
        capped = jnp.where(keep, masks, 0.0)
        masks_ref[...] = jnp.where(col_sums > CAPACITY, capped, masks)

    masks = masks_ref[...]
    expert_count = jnp.sum(masks, axis=0, keepdims=True)
    row_sum = jnp.maximum(jnp.sum(masks, axis=1, keepdims=True), EPS)
    masks_ref[...] = masks / row_sum

    total = jnp.maximum(jnp.sum(expert_count), EPS)
    usage = expert_count / total
    usage_ref[...] = usage
    target = 1.0 / E
    log_in = jnp.log(jnp.maximum(usage, EPS))
    kl = jnp.sum(target * (jnp.log(target) - log_in)) / E
    loss_ref[...] = jnp.full((1, 1), 0.01, jnp.float32) * kl


@functools.partial(jax.jit, static_argnames=())
def kernel(x, ln_w, ln_b, W1, b1, W2, b2):
    xf = x.reshape(N, H)
    lnw2 = ln_w.reshape(1, H)
    lnb2 = ln_b.reshape(1, H)
    b1_2 = b1.reshape(1, HH)
    b2_2 = b2.reshape(1, E)

    ew = pl.pallas_call(
        _mlp_kernel,
        grid=(N // TB, NBLK),
        in_specs=[
            pl.BlockSpec((TB, H), lambda i, j: (i, 0)),
            pl.BlockSpec((1, H), lambda i, j: (0, 0)),
            pl.BlockSpec((1, H), lambda i, j: (0, 0)),
            pl.BlockSpec((H, NB), lambda i, j: (0, j)),
            pl.BlockSpec((1, NB), lambda i, j: (0, j)),
            pl.BlockSpec((NB, E), lambda i, j: (j, 0)),
            pl.BlockSpec((1, E), lambda i, j: (0, 0)),
        ],
        out_specs=pl.BlockSpec((TB, E), lambda i, j: (i, 0)),
        out_shape=jax.ShapeDtypeStruct((N, E), jnp.float32),
        scratch_shapes=[
            pltpu.VMEM((TB, H), jnp.float32),
            pltpu.VMEM((TB, E), jnp.float32),
        ],
        compiler_params=pltpu.CompilerParams(
            dimension_semantics=("parallel", "arbitrary"),
        ),
    )(xf, lnw2, lnb2, W1, b1_2, W2, b2_2)

    masks, usage, loss = pl.pallas_call(
        _routing_kernel,
        out_shape=[
            jax.ShapeDtypeStruct((N, E), jnp.float32),
            jax.ShapeDtypeStruct((1, E), jnp.float32),
            jax.ShapeDtypeStruct((1, 1), jnp.float32),
        ],
    )(ew)

    return (ew.reshape(B, S, E), masks.reshape(B, S, E),
            loss.reshape(()), usage.reshape(E))


# speedup vs baseline: 1.1448x; 1.1448x over previous
import functools

import jax
import jax.numpy as jnp
import numpy as np
from jax.experimental import pallas as pl
from jax.experimental.pallas import tpu as pltpu

H = 4096
E = 16
TOP_K = 2
B = 4
S = 2048
N = B * S
HH = H // 2
TEMPERATURE = 0.7
CAPACITY_FACTOR = 2.0
EPS = 1e-6
CAPACITY = int(CAPACITY_FACTOR * B * S * TOP_K / E)

TB = 512
NB = 512
NBLK = HH // NB


def _mlp_kernel(x_ref, lnw_ref, lnb_ref, w1_ref, b1_ref, w2_ref, b2_ref,
                out_ref, xn_ref, acc_ref):
    j = pl.program_id(1)

    @pl.when(j == 0)
    def _ln():
        x = x_ref[...]
        mu = jnp.mean(x, axis=1, keepdims=True)
        xc = x - mu
        var = jnp.mean(xc * xc, axis=1, keepdims=True)
        inv = 1.0 / jnp.sqrt(var + 1e-5)
        xn_ref[...] = (xc * inv) * lnw_ref[...] + lnb_ref[...]

    h = jnp.dot(xn_ref[...], w1_ref[...], preferred_element_type=jnp.float32)
    h = h + b1_ref[...]
    h = 0.5 * h * (1.0 + jax.lax.erf(h * (1.0 / np.sqrt(2.0))))
    part = jnp.dot(h, w2_ref[...], preferred_element_type=jnp.float32)

    @pl.when(j == 0)
    def _init():
        acc_ref[...] = part

    @pl.when(j > 0)
    def _acc():
        acc_ref[...] = acc_ref[...] + part

    @pl.when(j == NBLK - 1)
    def _emit():
        ew = (acc_ref[...] + b2_ref[...]) / TEMPERATURE
        out_ref[...] = jnp.clip(ew, -50.0, 50.0)


def _routing_kernel(ew_ref, masks_ref, usage_ref, loss_ref):
    ew = ew_ref[...]
    idx_e = jax.lax.broadcasted_iota(jnp.int32, (N, E), 1)
    m1 = jnp.max(ew, axis=1, keepdims=True)
    i1 = jnp.min(jnp.where(ew == m1, idx_e, E), axis=1, keepdims=True)
    ew_m = jnp.where(idx_e == i1, -1e30, ew)
    m2 = jnp.max(ew_m, axis=1, keepdims=True)
    i2 = jnp.min(jnp.where(ew_m == m2, idx_e, E), axis=1, keepdims=True)
    e2 = jnp.exp(m2 - m1)
    denom = 1.0 + e2
    p1 = 1.0 / denom
    p2 = e2 / denom
    masks = jnp.where(idx_e == i1, p1, 0.0) + jnp.where(idx_e == i2, p2, 0.0)

    col_sums = jnp.sum(masks, axis=0, keepdims=True)
    masks_ref[...] = masks

    @pl.when(jnp.any(col_sums > CAPACITY))
    def _cap():
        vbits = jax.lax.bitcast_convert_type(masks, jnp.int32)
        thr = jnp.zeros((1, E), jnp.int32)
        for b in range(29, -1, -1):
            cand = thr | (1 << b)
            cnt = jnp.sum((vbits >= cand).astype(jnp.int32), axis=0,
                          keepdims=True)
            thr = jnp.where(cnt >= CAPACITY, cand, thr)
        count_gt = jnp.sum((vbits > thr).astype(jnp.int32), axis=0,
                           keepdims=True)
        rem = CAPACITY - count_gt
        eq = vbits == thr
        idx_n = jax.lax.broadcasted_iota(jnp.int32, (N, E), 0)
        jthr = jnp.zeros((1, E), jnp.int32)
        for b in range(12, -1, -1):
            cand = jthr | (1 << b)
            cnt = jnp.sum((eq & (idx_n <= cand)).astype(jnp.int32), axis=0,
                          keepdims=True)
            jthr = jnp.where(cnt <= rem, cand, jthr)
        keep = (vbits > thr) | (eq & (idx_n <= jthr))
        capped = jnp.where(keep, masks, 0.0)
        masks_ref[...] = jnp.where(col_sums > CAPACITY, capped, masks)

    masks = masks_ref[...]
    expert_count = jnp.sum(masks, axis=0, keepdims=True)
    row_sum = jnp.maximum(jnp.sum(masks, axis=1, keepdims=True), EPS)
    masks_ref[...] = masks / row_sum

    total = jnp.maximum(jnp.sum(expert_count), EPS)
    usage = expert_count / total
    usage_ref[...] = usage
    target = 1.0 / E
    log_in = jnp.log(jnp.maximum(usage, EPS))
    kl = jnp.sum(target * (jnp.log(target) - log_in)) / E
    loss_ref[...] = jnp.full((1, 1), 0.01, jnp.float32) * kl


@functools.partial(jax.jit, static_argnames=())
def kernel(x, ln_w, ln_b, W1, b1, W2, b2):
    xf = x.reshape(N, H)
    lnw2 = ln_w.reshape(1, H)
    lnb2 = ln_b.reshape(1, H)
    b1_2 = b1.reshape(1, HH)
    b2_2 = b2.reshape(1, E)

    ew = pl.pallas_call(
        _mlp_kernel,
        grid=(N // TB, NBLK),
        in_specs=[
            pl.BlockSpec((TB, H), lambda i, j: (i, 0)),
            pl.BlockSpec((1, H), lambda i, j: (0, 0)),
            pl.BlockSpec((1, H), lambda i, j: (0, 0)),
            pl.BlockSpec((H, NB), lambda i, j: (0, j)),
            pl.BlockSpec((1, NB), lambda i, j: (0, j)),
            pl.BlockSpec((NB, E), lambda i, j: (j, 0)),
            pl.BlockSpec((1, E), lambda i, j: (0, 0)),
        ],
        out_specs=pl.BlockSpec((TB, E), lambda i, j: (i, 0)),
        out_shape=jax.ShapeDtypeStruct((N, E), jnp.float32),
        scratch_shapes=[
            pltpu.VMEM((TB, H), jnp.float32),
            pltpu.VMEM((TB, E), jnp.float32),
        ],
        compiler_params=pltpu.CompilerParams(
            dimension_semantics=("parallel", "arbitrary"),
        ),
    )(xf, lnw2, lnb2, W1, b1_2, W2, b2_2)

    masks, usage, loss = pl.pallas_call(
        _routing_kernel,
        out_shape=[
            jax.ShapeDtypeStruct((N, E), jnp.float32),
            jax.ShapeDtypeStruct((1, E), jnp.float32),
            jax.ShapeDtypeStruct((1, 1), jnp.float32),
        ],
    )(ew)

    return (ew.reshape(B, S, E), masks.reshape(B, S, E),
            loss.reshape(()), usage.reshape(E))
